# Initial kernel scaffold; baseline (speedup 1.0000x reference)
import jax
import jax.numpy as jnp
from jax.experimental import pallas as pl


def kernel(inputs, embedding_weight):
    raise NotImplementedError("write your pallas kernel here")


# speedup vs baseline: 4.6629x; 4.6629x over previous
import functools

import jax
import jax.numpy as jnp
from jax import lax
from jax.experimental import pallas as pl
from jax.experimental.pallas import tpu as pltpu
from jax.experimental.pallas import tpu_sc as plsc

_D = 64
_NC = 2
_NS = 16
_NW = _NC * _NS
_CH = 128
_KG = 10
_ROWS_G = _CH * _KG


@functools.lru_cache(maxsize=None)
def _build_kernel(B: int):
    bpw = B // _NW
    nch = bpw // _CH
    ng = nch // _KG
    mesh = plsc.VectorSubcoreMesh(core_axis_name="c", subcore_axis_name="s")

    @functools.partial(
        pl.kernel,
        out_type=jax.ShapeDtypeStruct((B, _D), jnp.float32),
        mesh=mesh,
        scratch_types=[
            pltpu.VMEM((nch, _CH), jnp.int32),
            pltpu.VMEM((_ROWS_G, _D), jnp.float32),
            pltpu.SemaphoreType.DMA,
        ],
        compiler_params=pltpu.CompilerParams(use_tc_tiling_on_sc=False),
    )
    def k(idx_hbm, table_hbm, out_hbm, idx_v, rows_v, sem):
        wid = lax.axis_index("s") * _NC + lax.axis_index("c")
        base = wid * bpw
        pltpu.sync_copy(idx_hbm.at[wid], idx_v)

        def group(g, carry):
            copies = []
            for b in range(_KG):
                copies.append(pltpu.async_copy(
                    table_hbm.at[idx_v.at[g * _KG + b]],
                    rows_v.at[pl.ds(b * _CH, _CH)],
                    sem))
            for c in copies:
                c.wait()
            pltpu.sync_copy(rows_v,
                            out_hbm.at[pl.ds(base + g * _ROWS_G, _ROWS_G)])
            return carry

        lax.fori_loop(0, ng, group, 0)

    return k


def kernel(inputs, embedding_weight):
    batch, hist = inputs.shape
    B = batch * hist
    idx = inputs.astype(jnp.int32).reshape(_NW, B // _NW // _CH, _CH)
    out = _build_kernel(B)(idx, embedding_weight)
    return out.reshape(batch, hist, _D)


# speedup vs baseline: 4.6688x; 1.0013x over previous
import functools

import jax
import jax.numpy as jnp
from jax import lax
from jax.experimental import pallas as pl
from jax.experimental.pallas import tpu as pltpu
from jax.experimental.pallas import tpu_sc as plsc

_D = 64
_NC = 2
_NS = 16
_NW = _NC * _NS
_CH = 128
_KG = 5
_ROWS_G = _CH * _KG


@functools.lru_cache(maxsize=None)
def _build_kernel(B: int):
    bpw = B // _NW
    nch = bpw // _CH
    ng = nch // _KG
    mesh = plsc.VectorSubcoreMesh(core_axis_name="c", subcore_axis_name="s")

    @functools.partial(
        pl.kernel,
        out_type=jax.ShapeDtypeStruct((B, _D), jnp.float32),
        mesh=mesh,
        scratch_types=[
            pltpu.VMEM((nch, _CH), jnp.int32),
            pltpu.VMEM((2, _ROWS_G, _D), jnp.float32),
            pltpu.SemaphoreType.DMA,
            pltpu.SemaphoreType.DMA,
        ],
        compiler_params=pltpu.CompilerParams(use_tc_tiling_on_sc=False),
    )
    def k(idx_hbm, table_hbm, out_hbm, idx_v, rows_v, sem0, sem1):
        wid = lax.axis_index("s") * _NC + lax.axis_index("c")
        base = wid * bpw
        sems = (sem0, sem1)
        pltpu.sync_copy(idx_hbm.at[wid], idx_v)

        def fire(g, buf):
            for b in range(_KG):
                pltpu.async_copy(
                    table_hbm.at[idx_v.at[g * _KG + b]],
                    rows_v.at[buf].at[pl.ds(b * _CH, _CH)],
                    sems[buf])

        def drain_write(g, buf):
            for b in range(_KG):
                pltpu.make_async_copy(
                    table_hbm.at[idx_v.at[g * _KG + b]],
                    rows_v.at[buf].at[pl.ds(b * _CH, _CH)],
                    sems[buf]).wait()
            pltpu.sync_copy(rows_v.at[buf],
                            out_hbm.at[pl.ds(base + g * _ROWS_G, _ROWS_G)])

        fire(0, 0)

        def pair(i, carry):
            g0 = i * 2
            fire(g0 + 1, 1)
            drain_write(g0, 0)

            @pl.when(g0 + 2 < ng)
            def _():
                fire(g0 + 2, 0)

            drain_write(g0 + 1, 1)
            return carry

        lax.fori_loop(0, ng // 2, pair, 0)

    return k


def kernel(inputs, embedding_weight):
    batch, hist = inputs.shape
    B = batch * hist
    idx = inputs.astype(jnp.int32).reshape(_NW, B // _NW // _CH, _CH)
    out = _build_kernel(B)(idx, embedding_weight)
    return out.reshape(batch, hist, _D)
